# baseline (device time: 329515 ns/iter reference)
import jax
import jax.numpy as jnp
from jax import lax
from jax.experimental import pallas as pl
from jax.experimental.pallas import tpu as pltpu

N_DEV = 4
M = 4096
D = 4096
CH = M // N_DEV
Q = CH // 4

FLOWS = ((0 * Q, +1), (2 * Q, -1), (1 * Q, +1), (3 * Q, -1))
N_FLOW = len(FLOWS)


def kernel(partial, resid, gamma):
    g = gamma.reshape(1, D)

    def body(x_ref, resid_ref, g_ref, out_ref,
             acc, recv, stage,
             send_sems, recv_sems, credit_sems, load_sems, store_sems):
        my = lax.axis_index("i")
        left = (my - 1) % N_DEV
        right = (my + 1) % N_DEV

        barrier = pltpu.get_barrier_semaphore()
        for nbr in (left, right):
            pl.semaphore_signal(barrier, inc=1, device_id=(nbr,),
                                device_id_type=pl.DeviceIdType.MESH)
        pl.semaphore_wait(barrier, 2)

        def send_flow(f):
            off, sgn = FLOWS[f]
            rdma = pltpu.make_async_remote_copy(
                src_ref=acc.at[pl.ds(off, Q), :],
                dst_ref=recv.at[pl.ds(off, Q), :],
                send_sem=send_sems.at[f], recv_sem=recv_sems.at[f],
                device_id=(right if sgn > 0 else left,),
                device_id_type=pl.DeviceIdType.MESH)
            rdma.start()
            return rdma

        def send_wait(f):
            off, sgn = FLOWS[f]
            pltpu.make_async_remote_copy(
                src_ref=acc.at[pl.ds(off, Q), :],
                dst_ref=recv.at[pl.ds(off, Q), :],
                send_sem=send_sems.at[f], recv_sem=recv_sems.at[f],
                device_id=(right if sgn > 0 else left,),
                device_id_type=pl.DeviceIdType.MESH).wait_send()

        def recv_wait(f):
            off, sgn = FLOWS[f]
            pltpu.make_async_remote_copy(
                src_ref=acc.at[pl.ds(off, Q), :],
                dst_ref=recv.at[pl.ds(off, Q), :],
                send_sem=send_sems.at[f], recv_sem=recv_sems.at[f],
                device_id=(right if sgn > 0 else left,),
                device_id_type=pl.DeviceIdType.MESH).wait_recv()

        def load_flow(f, chunk, src_ref=None):
            off, _ = FLOWS[f]
            if src_ref is None:
                src = x_ref.at[0, pl.ds(chunk * CH + off, Q), :]
            else:
                src = src_ref.at[pl.ds(chunk * CH + off, Q), :]
            cp = pltpu.make_async_copy(
                src, stage.at[pl.ds(off, Q), :], load_sems.at[f])
            cp.start()
            return cp

        def credit_source(f):
            _, sgn = FLOWS[f]
            pl.semaphore_signal(credit_sems.at[f], inc=1,
                                device_id=(left if sgn > 0 else right,),
                                device_id_type=pl.DeviceIdType.MESH)

        def store_flow(f, chunk):
            off, _ = FLOWS[f]
            cp = pltpu.make_async_copy(
                stage.at[pl.ds(off, Q), :],
                out_ref.at[pl.ds(chunk * CH + off, Q), :], store_sems.at[f])
            cp.start()
            return cp

        def rs_chunk(f, s):
            _, sgn = FLOWS[f]
            return (my - s - 1) % N_DEV if sgn > 0 else (my + s + 1) % N_DEV

        def own_chunk(f):
            _, sgn = FLOWS[f]
            return (my + 1) % N_DEV if sgn > 0 else (my - 1) % N_DEV

        def ag_chunk(f, h):
            _, sgn = FLOWS[f]
            return (my - h) % N_DEV if sgn > 0 else (my + h) % N_DEV

        for f in range(N_FLOW):
            load_flow(f, my)
        for f in range(N_FLOW):
            off, _ = FLOWS[f]
            pltpu.make_async_copy(
                x_ref.at[0, pl.ds(my * CH + off, Q), :],
                stage.at[pl.ds(off, Q), :], load_sems.at[f]).wait()
            acc[pl.ds(off, Q), :] = stage[pl.ds(off, Q), :].astype(jnp.bfloat16)
            send_flow(f)
            load_flow(f, rs_chunk(f, 0))

        for s in range(N_DEV - 1):
            for f in range(N_FLOW):
                off, _ = FLOWS[f]
                recv_wait(f)
                send_wait(f)
                pltpu.make_async_copy(
                    x_ref.at[0, pl.ds(0, Q), :],
                    stage.at[pl.ds(off, Q), :], load_sems.at[f]).wait()
                acc[pl.ds(off, Q), :] = (
                    recv[pl.ds(off, Q), :]
                    + stage[pl.ds(off, Q), :].astype(jnp.bfloat16))
                credit_source(f)
                if s < N_DEV - 2:
                    pl.semaphore_wait(credit_sems.at[f], 1)
                    send_flow(f)
                    load_flow(f, rs_chunk(f, s + 1))
                else:
                    load_flow(f, own_chunk(f), src_ref=resid_ref)

        for f in range(N_FLOW):
            off, _ = FLOWS[f]
            pltpu.make_async_copy(
                resid_ref.at[pl.ds(0, Q), :],
                stage.at[pl.ds(off, Q), :], load_sems.at[f]).wait()
            y = (stage[pl.ds(off, Q), :]
                 + acc[pl.ds(off, Q), :].astype(jnp.float32))
            rms = jnp.sqrt(jnp.mean(y * y, axis=-1, keepdims=True) + 1e-6)
            res = y / rms * g_ref[...]
            stage[pl.ds(off, Q), :] = res
            acc[pl.ds(off, Q), :] = res.astype(jnp.bfloat16)
            pl.semaphore_wait(credit_sems.at[f], 1)
            send_flow(f)
            store_flow(f, own_chunk(f))

        for h in range(N_DEV - 1):
            for f in range(N_FLOW):
                off, _ = FLOWS[f]
                recv_wait(f)
                send_wait(f)
                if h < N_DEV - 2:
                    acc[pl.ds(off, Q), :] = recv[pl.ds(off, Q), :]
                    credit_source(f)
                    pl.semaphore_wait(credit_sems.at[f], 1)
                    send_flow(f)
                    src = acc
                else:
                    src = recv
                pltpu.make_async_copy(
                    stage.at[pl.ds(off, Q), :],
                    out_ref.at[pl.ds(0, Q), :], store_sems.at[f]).wait()
                stage[pl.ds(off, Q), :] = (
                    src[pl.ds(off, Q), :].astype(jnp.float32))
                store_flow(f, ag_chunk(f, h))
        for f in range(N_FLOW):
            off, _ = FLOWS[f]
            pltpu.make_async_copy(
                stage.at[pl.ds(off, Q), :],
                out_ref.at[pl.ds(0, Q), :], store_sems.at[f]).wait()

    return pl.pallas_call(
        body,
        out_shape=jax.ShapeDtypeStruct((M, D), jnp.float32),
        in_specs=[
            pl.BlockSpec(memory_space=pl.ANY),
            pl.BlockSpec(memory_space=pl.ANY),
            pl.BlockSpec(memory_space=pltpu.VMEM),
        ],
        out_specs=pl.BlockSpec(memory_space=pl.ANY),
        scratch_shapes=[
            pltpu.VMEM((CH, D), jnp.bfloat16),
            pltpu.VMEM((CH, D), jnp.bfloat16),
            pltpu.VMEM((CH, D), jnp.float32),
            pltpu.SemaphoreType.DMA((N_FLOW,)),
            pltpu.SemaphoreType.DMA((N_FLOW,)),
            pltpu.SemaphoreType.REGULAR((N_FLOW,)),
            pltpu.SemaphoreType.DMA((N_FLOW,)),
            pltpu.SemaphoreType.DMA((N_FLOW,)),
        ],
        compiler_params=pltpu.CompilerParams(
            collective_id=0, vmem_limit_bytes=60 * 1024 * 1024
        ),
    )(partial, resid, g)


# device time: 329204 ns/iter; 1.0009x vs baseline; 1.0009x over previous
import jax
import jax.numpy as jnp
from jax import lax
from jax.experimental import pallas as pl
from jax.experimental.pallas import tpu as pltpu

N_DEV = 4
M = 4096
D = 4096
CH = M // N_DEV
Q = CH // 4

FLOWS = ((0 * Q, +1), (2 * Q, -1), (1 * Q, +1), (3 * Q, -1))
N_FLOW = len(FLOWS)


def kernel(partial, resid, gamma):
    g = gamma.reshape(1, D)

    def body(x_ref, resid_ref, g_ref, out_ref,
             acc, recv, stage,
             send_sems, recv_sems, credit_sems, load_sems, store_sems):
        my = lax.axis_index("i")
        left = (my - 1) % N_DEV
        right = (my + 1) % N_DEV

        def send_flow(f):
            off, sgn = FLOWS[f]
            rdma = pltpu.make_async_remote_copy(
                src_ref=acc.at[pl.ds(off, Q), :],
                dst_ref=recv.at[pl.ds(off, Q), :],
                send_sem=send_sems.at[f], recv_sem=recv_sems.at[f],
                device_id=(right if sgn > 0 else left,),
                device_id_type=pl.DeviceIdType.MESH)
            rdma.start()
            return rdma

        def send_wait(f):
            off, sgn = FLOWS[f]
            pltpu.make_async_remote_copy(
                src_ref=acc.at[pl.ds(off, Q), :],
                dst_ref=recv.at[pl.ds(off, Q), :],
                send_sem=send_sems.at[f], recv_sem=recv_sems.at[f],
                device_id=(right if sgn > 0 else left,),
                device_id_type=pl.DeviceIdType.MESH).wait_send()

        def recv_wait(f):
            off, sgn = FLOWS[f]
            pltpu.make_async_remote_copy(
                src_ref=acc.at[pl.ds(off, Q), :],
                dst_ref=recv.at[pl.ds(off, Q), :],
                send_sem=send_sems.at[f], recv_sem=recv_sems.at[f],
                device_id=(right if sgn > 0 else left,),
                device_id_type=pl.DeviceIdType.MESH).wait_recv()

        def load_flow(f, chunk, src_ref=None):
            off, _ = FLOWS[f]
            if src_ref is None:
                src = x_ref.at[0, pl.ds(chunk * CH + off, Q), :]
            else:
                src = src_ref.at[pl.ds(chunk * CH + off, Q), :]
            cp = pltpu.make_async_copy(
                src, stage.at[pl.ds(off, Q), :], load_sems.at[f])
            cp.start()
            return cp

        def credit_source(f):
            _, sgn = FLOWS[f]
            pl.semaphore_signal(credit_sems.at[f], inc=1,
                                device_id=(left if sgn > 0 else right,),
                                device_id_type=pl.DeviceIdType.MESH)

        def store_flow(f, chunk):
            off, _ = FLOWS[f]
            cp = pltpu.make_async_copy(
                stage.at[pl.ds(off, Q), :],
                out_ref.at[pl.ds(chunk * CH + off, Q), :], store_sems.at[f])
            cp.start()
            return cp

        def rs_chunk(f, s):
            _, sgn = FLOWS[f]
            return (my - s - 1) % N_DEV if sgn > 0 else (my + s + 1) % N_DEV

        def own_chunk(f):
            _, sgn = FLOWS[f]
            return (my + 1) % N_DEV if sgn > 0 else (my - 1) % N_DEV

        def ag_chunk(f, h):
            _, sgn = FLOWS[f]
            return (my - h) % N_DEV if sgn > 0 else (my + h) % N_DEV

        for f in range(N_FLOW):
            load_flow(f, my)

        barrier = pltpu.get_barrier_semaphore()
        for nbr in (left, right):
            pl.semaphore_signal(barrier, inc=1, device_id=(nbr,),
                                device_id_type=pl.DeviceIdType.MESH)
        pl.semaphore_wait(barrier, 2)

        for f in range(N_FLOW):
            off, _ = FLOWS[f]
            pltpu.make_async_copy(
                x_ref.at[0, pl.ds(my * CH + off, Q), :],
                stage.at[pl.ds(off, Q), :], load_sems.at[f]).wait()
            acc[pl.ds(off, Q), :] = stage[pl.ds(off, Q), :].astype(jnp.bfloat16)
            send_flow(f)
            load_flow(f, rs_chunk(f, 0))

        for s in range(N_DEV - 1):
            for f in range(N_FLOW):
                off, _ = FLOWS[f]
                recv_wait(f)
                send_wait(f)
                pltpu.make_async_copy(
                    x_ref.at[0, pl.ds(0, Q), :],
                    stage.at[pl.ds(off, Q), :], load_sems.at[f]).wait()
                acc[pl.ds(off, Q), :] = (
                    recv[pl.ds(off, Q), :]
                    + stage[pl.ds(off, Q), :].astype(jnp.bfloat16))
                credit_source(f)
                if s < N_DEV - 2:
                    pl.semaphore_wait(credit_sems.at[f], 1)
                    send_flow(f)
                    load_flow(f, rs_chunk(f, s + 1))
                else:
                    load_flow(f, own_chunk(f), src_ref=resid_ref)

        for f in range(N_FLOW):
            off, _ = FLOWS[f]
            pltpu.make_async_copy(
                resid_ref.at[pl.ds(0, Q), :],
                stage.at[pl.ds(off, Q), :], load_sems.at[f]).wait()
            y = (stage[pl.ds(off, Q), :]
                 + acc[pl.ds(off, Q), :].astype(jnp.float32))
            rms = jnp.sqrt(jnp.mean(y * y, axis=-1, keepdims=True) + 1e-6)
            res = y / rms * g_ref[...]
            stage[pl.ds(off, Q), :] = res
            acc[pl.ds(off, Q), :] = res.astype(jnp.bfloat16)
            pl.semaphore_wait(credit_sems.at[f], 1)
            send_flow(f)
            store_flow(f, own_chunk(f))

        for h in range(N_DEV - 1):
            for f in range(N_FLOW):
                off, _ = FLOWS[f]
                recv_wait(f)
                send_wait(f)
                if h < N_DEV - 2:
                    acc[pl.ds(off, Q), :] = recv[pl.ds(off, Q), :]
                    credit_source(f)
                    pl.semaphore_wait(credit_sems.at[f], 1)
                    send_flow(f)
                    src = acc
                else:
                    src = recv
                pltpu.make_async_copy(
                    stage.at[pl.ds(off, Q), :],
                    out_ref.at[pl.ds(0, Q), :], store_sems.at[f]).wait()
                stage[pl.ds(off, Q), :] = (
                    src[pl.ds(off, Q), :].astype(jnp.float32))
                store_flow(f, ag_chunk(f, h))
        for f in range(N_FLOW):
            off, _ = FLOWS[f]
            pltpu.make_async_copy(
                stage.at[pl.ds(off, Q), :],
                out_ref.at[pl.ds(0, Q), :], store_sems.at[f]).wait()

    return pl.pallas_call(
        body,
        out_shape=jax.ShapeDtypeStruct((M, D), jnp.float32),
        in_specs=[
            pl.BlockSpec(memory_space=pl.ANY),
            pl.BlockSpec(memory_space=pl.ANY),
            pl.BlockSpec(memory_space=pltpu.VMEM),
        ],
        out_specs=pl.BlockSpec(memory_space=pl.ANY),
        scratch_shapes=[
            pltpu.VMEM((CH, D), jnp.bfloat16),
            pltpu.VMEM((CH, D), jnp.bfloat16),
            pltpu.VMEM((CH, D), jnp.float32),
            pltpu.SemaphoreType.DMA((N_FLOW,)),
            pltpu.SemaphoreType.DMA((N_FLOW,)),
            pltpu.SemaphoreType.REGULAR((N_FLOW,)),
            pltpu.SemaphoreType.DMA((N_FLOW,)),
            pltpu.SemaphoreType.DMA((N_FLOW,)),
        ],
        compiler_params=pltpu.CompilerParams(
            collective_id=0, vmem_limit_bytes=60 * 1024 * 1024
        ),
    )(partial, resid, g)


# device time: 325850 ns/iter; 1.0112x vs baseline; 1.0103x over previous
import jax
import jax.numpy as jnp
from jax import lax
from jax.experimental import pallas as pl
from jax.experimental.pallas import tpu as pltpu

N_DEV = 4
M = 4096
D = 4096
CH = M // N_DEV
Q = CH // 8

FLOWS = tuple(
    pair
    for k in range(4)
    for pair in ((k * Q, +1), ((k + 4) * Q, -1))
)
N_FLOW = len(FLOWS)


def kernel(partial, resid, gamma):
    g = gamma.reshape(1, D)

    def body(x_ref, resid_ref, g_ref, out_ref,
             acc, recv, stage,
             send_sems, recv_sems, credit_sems, load_sems, store_sems):
        my = lax.axis_index("i")
        left = (my - 1) % N_DEV
        right = (my + 1) % N_DEV

        def send_flow(f):
            off, sgn = FLOWS[f]
            rdma = pltpu.make_async_remote_copy(
                src_ref=acc.at[pl.ds(off, Q), :],
                dst_ref=recv.at[pl.ds(off, Q), :],
                send_sem=send_sems.at[f], recv_sem=recv_sems.at[f],
                device_id=(right if sgn > 0 else left,),
                device_id_type=pl.DeviceIdType.MESH)
            rdma.start()
            return rdma

        def send_wait(f):
            off, sgn = FLOWS[f]
            pltpu.make_async_remote_copy(
                src_ref=acc.at[pl.ds(off, Q), :],
                dst_ref=recv.at[pl.ds(off, Q), :],
                send_sem=send_sems.at[f], recv_sem=recv_sems.at[f],
                device_id=(right if sgn > 0 else left,),
                device_id_type=pl.DeviceIdType.MESH).wait_send()

        def recv_wait(f):
            off, sgn = FLOWS[f]
            pltpu.make_async_remote_copy(
                src_ref=acc.at[pl.ds(off, Q), :],
                dst_ref=recv.at[pl.ds(off, Q), :],
                send_sem=send_sems.at[f], recv_sem=recv_sems.at[f],
                device_id=(right if sgn > 0 else left,),
                device_id_type=pl.DeviceIdType.MESH).wait_recv()

        def load_flow(f, chunk, src_ref=None):
            off, _ = FLOWS[f]
            if src_ref is None:
                src = x_ref.at[0, pl.ds(chunk * CH + off, Q), :]
            else:
                src = src_ref.at[pl.ds(chunk * CH + off, Q), :]
            cp = pltpu.make_async_copy(
                src, stage.at[pl.ds(off, Q), :], load_sems.at[f])
            cp.start()
            return cp

        def credit_source(f):
            _, sgn = FLOWS[f]
            pl.semaphore_signal(credit_sems.at[f], inc=1,
                                device_id=(left if sgn > 0 else right,),
                                device_id_type=pl.DeviceIdType.MESH)

        def store_flow(f, chunk):
            off, _ = FLOWS[f]
            cp = pltpu.make_async_copy(
                stage.at[pl.ds(off, Q), :],
                out_ref.at[pl.ds(chunk * CH + off, Q), :], store_sems.at[f])
            cp.start()
            return cp

        def rs_chunk(f, s):
            _, sgn = FLOWS[f]
            return (my - s - 1) % N_DEV if sgn > 0 else (my + s + 1) % N_DEV

        def own_chunk(f):
            _, sgn = FLOWS[f]
            return (my + 1) % N_DEV if sgn > 0 else (my - 1) % N_DEV

        def ag_chunk(f, h):
            _, sgn = FLOWS[f]
            return (my - h) % N_DEV if sgn > 0 else (my + h) % N_DEV

        for f in range(N_FLOW):
            load_flow(f, my)

        barrier = pltpu.get_barrier_semaphore()
        for nbr in (left, right):
            pl.semaphore_signal(barrier, inc=1, device_id=(nbr,),
                                device_id_type=pl.DeviceIdType.MESH)
        pl.semaphore_wait(barrier, 2)

        for f in range(N_FLOW):
            off, _ = FLOWS[f]
            pltpu.make_async_copy(
                x_ref.at[0, pl.ds(my * CH + off, Q), :],
                stage.at[pl.ds(off, Q), :], load_sems.at[f]).wait()
            acc[pl.ds(off, Q), :] = stage[pl.ds(off, Q), :].astype(jnp.bfloat16)
            send_flow(f)
            load_flow(f, rs_chunk(f, 0))

        for s in range(N_DEV - 1):
            for f in range(N_FLOW):
                off, _ = FLOWS[f]
                recv_wait(f)
                send_wait(f)
                pltpu.make_async_copy(
                    x_ref.at[0, pl.ds(0, Q), :],
                    stage.at[pl.ds(off, Q), :], load_sems.at[f]).wait()
                acc[pl.ds(off, Q), :] = (
                    recv[pl.ds(off, Q), :]
                    + stage[pl.ds(off, Q), :].astype(jnp.bfloat16))
                credit_source(f)
                if s < N_DEV - 2:
                    pl.semaphore_wait(credit_sems.at[f], 1)
                    send_flow(f)
                    load_flow(f, rs_chunk(f, s + 1))
                else:
                    load_flow(f, own_chunk(f), src_ref=resid_ref)

        for f in range(N_FLOW):
            off, _ = FLOWS[f]
            pltpu.make_async_copy(
                resid_ref.at[pl.ds(0, Q), :],
                stage.at[pl.ds(off, Q), :], load_sems.at[f]).wait()
            y = (stage[pl.ds(off, Q), :]
                 + acc[pl.ds(off, Q), :].astype(jnp.float32))
            rms = jnp.sqrt(jnp.mean(y * y, axis=-1, keepdims=True) + 1e-6)
            res = y / rms * g_ref[...]
            stage[pl.ds(off, Q), :] = res
            acc[pl.ds(off, Q), :] = res.astype(jnp.bfloat16)
            pl.semaphore_wait(credit_sems.at[f], 1)
            send_flow(f)
            store_flow(f, own_chunk(f))

        for h in range(N_DEV - 1):
            for f in range(N_FLOW):
                off, _ = FLOWS[f]
                recv_wait(f)
                send_wait(f)
                if h < N_DEV - 2:
                    acc[pl.ds(off, Q), :] = recv[pl.ds(off, Q), :]
                    credit_source(f)
                    pl.semaphore_wait(credit_sems.at[f], 1)
                    send_flow(f)
                    src = acc
                else:
                    src = recv
                pltpu.make_async_copy(
                    stage.at[pl.ds(off, Q), :],
                    out_ref.at[pl.ds(0, Q), :], store_sems.at[f]).wait()
                stage[pl.ds(off, Q), :] = (
                    src[pl.ds(off, Q), :].astype(jnp.float32))
                store_flow(f, ag_chunk(f, h))
        for f in range(N_FLOW):
            off, _ = FLOWS[f]
            pltpu.make_async_copy(
                stage.at[pl.ds(off, Q), :],
                out_ref.at[pl.ds(0, Q), :], store_sems.at[f]).wait()

    return pl.pallas_call(
        body,
        out_shape=jax.ShapeDtypeStruct((M, D), jnp.float32),
        in_specs=[
            pl.BlockSpec(memory_space=pl.ANY),
            pl.BlockSpec(memory_space=pl.ANY),
            pl.BlockSpec(memory_space=pltpu.VMEM),
        ],
        out_specs=pl.BlockSpec(memory_space=pl.ANY),
        scratch_shapes=[
            pltpu.VMEM((CH, D), jnp.bfloat16),
            pltpu.VMEM((CH, D), jnp.bfloat16),
            pltpu.VMEM((CH, D), jnp.float32),
            pltpu.SemaphoreType.DMA((N_FLOW,)),
            pltpu.SemaphoreType.DMA((N_FLOW,)),
            pltpu.SemaphoreType.REGULAR((N_FLOW,)),
            pltpu.SemaphoreType.DMA((N_FLOW,)),
            pltpu.SemaphoreType.DMA((N_FLOW,)),
        ],
        compiler_params=pltpu.CompilerParams(
            collective_id=0, vmem_limit_bytes=60 * 1024 * 1024
        ),
    )(partial, resid, g)
